# baseline (speedup 1.0000x reference)
import functools

import jax
import jax.numpy as jnp
from jax import lax
from jax.experimental import pallas as pl
from jax.experimental.pallas import tpu as pltpu
from jax.experimental.pallas import tpu_sc as plsc

HIST = 50
BATCH = 16384
DIM = 64
LB = 128
BT_PER_UNIT = 2
UNIT = BT_PER_UNIT * LB


@functools.cache
def _build():
    info = plsc.get_sparse_core_info()
    nc, ns = info.num_cores, info.num_subcores
    nw = nc * ns
    n_bt = BATCH // LB
    n_units = HIST * (n_bt // BT_PER_UNIT)
    u_per_w = n_units // nw
    assert u_per_w * nw == n_units and u_per_w % 2 == 0

    mesh = plsc.VectorSubcoreMesh(core_axis_name="c", subcore_axis_name="s")

    @functools.partial(
        pl.kernel,
        mesh=mesh,
        out_type=jax.ShapeDtypeStruct((HIST, DIM // 8, n_bt, 8, LB), jnp.float32),
        scratch_types=[
            pltpu.VMEM((n_units // nw, BT_PER_UNIT, LB), jnp.int32),
            pltpu.VMEM((2, UNIT, DIM), jnp.float32),
            pltpu.VMEM((2, DIM // 8, BT_PER_UNIT, 8, LB), jnp.float32),
            pltpu.SemaphoreType.DMA,
            pltpu.SemaphoreType.DMA,
        ],
        compiler_params=pltpu.CompilerParams(
            use_tc_tiling_on_sc=False, needs_layout_passes=False
        ),
    )
    def k(idx_hbm, table_hbm, out_hbm, idx_v, gath_v, tr_v, gsem, osem):
        wid = lax.axis_index("s") * nc + lax.axis_index("c")
        u0 = wid * u_per_w
        lane16 = lax.iota(jnp.int32, 16)
        zeros16 = lane16 * 0

        def unit_hb(u):
            h = u // (n_bt // BT_PER_UNIT)
            bt0 = (u % (n_bt // BT_PER_UNIT)) * BT_PER_UNIT
            return h, bt0

        def fire(u, b):
            for j in range(BT_PER_UNIT):
                pltpu.async_copy(
                    table_hbm.at[idx_v.at[u - u0, j]],
                    gath_v.at[b, pl.ds(j * LB, LB)],
                    gsem,
                )

        def drain_gather(b):
            for j in range(BT_PER_UNIT):
                pltpu.make_async_copy(
                    table_hbm.at[idx_v.at[0, j]],
                    gath_v.at[b, pl.ds(j * LB, LB)],
                    gsem,
                ).wait()

        groups = [(btr, g) for btr in range(BT_PER_UNIT) for g in range(LB // 16)]
        rows_l = [lane16 + (btr * LB + g * 16) for btr, g in groups]

        def transpose(b):
            @pl.loop(0, DIM, unroll=2)
            def _d(d):
                dt = d // 8
                sd = d % 8
                col = zeros16 + d
                vecs = [
                    plsc.load_gather(gath_v.at[b], [rows, col]) for rows in rows_l
                ]
                for gi, (btr, g) in enumerate(groups):
                    tr_v[b, dt, btr, sd, pl.ds(g * 16, 16)] = vecs[gi]

        def fire_out(u, b):
            h, bt0 = unit_hb(u)
            for dt in range(DIM // 8):
                pltpu.async_copy(
                    tr_v.at[b, dt],
                    out_hbm.at[h, dt, pl.ds(bt0, BT_PER_UNIT)],
                    osem,
                )

        def drain_out(b):
            for dt in range(DIM // 8):
                pltpu.make_async_copy(
                    tr_v.at[b, dt],
                    out_hbm.at[0, dt, pl.ds(0, BT_PER_UNIT)],
                    osem,
                ).wait()

        pltpu.sync_copy(idx_hbm.at[pl.ds(u0, u_per_w)], idx_v)

        fire(u0, 0)
        fire(u0 + 1, 1)
        drain_gather(0)
        transpose(0)
        fire_out(u0, 0)

        @pl.loop(0, (u_per_w - 2) // 2)
        def _pair(p):
            g = u0 + 1 + 2 * p
            for db in range(2):
                b = (1 + db) % 2
                drain_out(1 - b)
                fire(g + db + 1, 1 - b)
                drain_gather(b)
                transpose(b)
                fire_out(g + db, b)

        drain_out(0)
        drain_gather(1)
        transpose(1)
        fire_out(u0 + u_per_w - 1, 1)
        drain_out(1)

    return k


def kernel(x, weight):
    idx = x.astype(jnp.int32).T.reshape(-1, BT_PER_UNIT, LB)
    out5 = _build()(idx, weight)
    return out5.transpose(2, 4, 0, 1, 3).reshape(BATCH, HIST, DIM)


# speedup vs baseline: 2.0849x; 2.0849x over previous
import functools

import jax
import jax.numpy as jnp
from jax import lax
from jax.experimental import pallas as pl
from jax.experimental.pallas import tpu as pltpu
from jax.experimental.pallas import tpu_sc as plsc

HIST = 50
BATCH = 16384
DIM = 64
LB = 128
BT_PER_UNIT = 2
UNIT = BT_PER_UNIT * LB


@functools.cache
def _build():
    info = plsc.get_sparse_core_info()
    nc, ns = info.num_cores, info.num_subcores
    nw = nc * ns
    n_bt = BATCH // LB
    n_units = HIST * (n_bt // BT_PER_UNIT)
    u_per_w = n_units // nw
    assert u_per_w * nw == n_units and u_per_w % 2 == 0

    mesh = plsc.VectorSubcoreMesh(core_axis_name="c", subcore_axis_name="s")

    @functools.partial(
        pl.kernel,
        mesh=mesh,
        out_type=jax.ShapeDtypeStruct(
            (HIST, DIM // 8, n_bt // BT_PER_UNIT, BT_PER_UNIT * 8 * LB), jnp.float32
        ),
        scratch_types=[
            pltpu.VMEM((n_units // nw, BT_PER_UNIT, LB), jnp.int32),
            pltpu.VMEM((2, UNIT, DIM), jnp.float32),
            pltpu.VMEM((2, UNIT * DIM), jnp.float32),
            pltpu.SemaphoreType.DMA,
            pltpu.SemaphoreType.DMA,
        ],
        compiler_params=pltpu.CompilerParams(
            use_tc_tiling_on_sc=False, needs_layout_passes=False
        ),
    )
    def k(idx_hbm, table_hbm, out_hbm, idx_v, gath_v, tr_v, gsem, osem):
        wid = lax.axis_index("s") * nc + lax.axis_index("c")
        u0 = wid * u_per_w
        lane16 = lax.iota(jnp.int32, 16)
        zeros16 = lane16 * 0

        def unit_hb(u):
            h = u // (n_bt // BT_PER_UNIT)
            btg = u % (n_bt // BT_PER_UNIT)
            return h, btg

        def fire(u, b):
            for j in range(BT_PER_UNIT):
                pltpu.async_copy(
                    table_hbm.at[idx_v.at[u - u0, j]],
                    gath_v.at[b, pl.ds(j * LB, LB)],
                    gsem,
                )

        def drain_gather(b):
            for j in range(BT_PER_UNIT):
                pltpu.make_async_copy(
                    table_hbm.at[idx_v.at[0, j]],
                    gath_v.at[b, pl.ds(j * LB, LB)],
                    gsem,
                ).wait()

        wrap = [(lane16 + k) & 15 for k in range(16)]
        ldw = [lane16 * DIM + w for w in wrap]
        stw = [
            lane16 + (w >> 3) * (BT_PER_UNIT * 8 * LB) + (w & 7) * LB for w in wrap
        ]

        def transpose(b):
            @pl.loop(0, 16, unroll=2)
            def _jb(jb):
                j0 = jb * 16
                ld_base = j0 * DIM
                st_base = (jb // 8) * (8 * LB) + (jb % 8) * 16
                for d0i in range(4):
                    d0 = d0i * 16
                    lb_ = zeros16 + (ld_base + d0)
                    sb_ = zeros16 + (st_base + (d0 // 8) * (BT_PER_UNIT * 8 * LB))
                    vecs = [
                        plsc.load_gather(gath_v.at[b], [zeros16, ldw[k] + lb_])
                        for k in range(16)
                    ]
                    for k in range(16):
                        plsc.store_scatter(tr_v.at[b], [stw[k] + sb_], vecs[k])

        SLAB = BT_PER_UNIT * 8 * LB

        def fire_out(u, b):
            h, btg = unit_hb(u)
            for dt in range(DIM // 8):
                pltpu.async_copy(
                    tr_v.at[b, pl.ds(dt * SLAB, SLAB)],
                    out_hbm.at[h, dt, btg],
                    osem,
                )

        def drain_out(b):
            for dt in range(DIM // 8):
                pltpu.make_async_copy(
                    tr_v.at[b, pl.ds(dt * SLAB, SLAB)],
                    out_hbm.at[0, dt, 0],
                    osem,
                ).wait()

        pltpu.sync_copy(idx_hbm.at[pl.ds(u0, u_per_w)], idx_v)

        fire(u0, 0)
        fire(u0 + 1, 1)
        drain_gather(0)
        transpose(0)
        fire_out(u0, 0)

        @pl.loop(0, (u_per_w - 2) // 2)
        def _pair(p):
            g = u0 + 1 + 2 * p
            for db in range(2):
                b = (1 + db) % 2
                drain_out(1 - b)
                fire(g + db + 1, 1 - b)
                drain_gather(b)
                transpose(b)
                fire_out(g + db, b)

        drain_out(0)
        drain_gather(1)
        transpose(1)
        fire_out(u0 + u_per_w - 1, 1)
        drain_out(1)

    return k


def kernel(x, weight):
    idx = x.astype(jnp.int32).T.reshape(-1, BT_PER_UNIT, LB)
    out4 = _build()(idx, weight)
    out5 = out4.reshape(HIST, DIM // 8, BATCH // LB, 8, LB)
    return out5.transpose(2, 4, 0, 1, 3).reshape(BATCH, HIST, DIM)
